# baseline (speedup 1.0000x reference)
import jax
import jax.numpy as jnp
from jax.experimental import pallas as pl
from jax.experimental.pallas import tpu as pltpu

B, S, H = 1, 2048, 1024
E, K = 8, 2
I = 256
G = 128
TEMP = 1.0
PAIR = 2
NP = E // PAIR
NT = 2
S2 = S // NT

_DN = (((1,), (1,)), ((), ()))


def _dmlp(x, wg, wu):
    xg = jax.lax.dot_general(x, wg, _DN, preferred_element_type=jnp.float32)
    xu = jax.lax.dot_general(x, wu, _DN, preferred_element_type=jnp.float32)
    return jnp.where(xg >= 0, xg, 0.01 * xg) * xu


def _moe_body(x_ref, g_ref, wt_ref, wgene_ref, Wsg_ref, Wsu_ref, Wsd_ref,
              Wg_ref, Wu_ref, Wd_ref, out_ref, w9_scr, xb_scr, acc_scr):
    p = pl.program_id(0)
    t = pl.program_id(1)

    @pl.when(p == 0)
    def _router_and_shared():
        xb_scr[t] = x_ref[...].astype(jnp.bfloat16)
        x = xb_scr[t]
        logits_h = jax.lax.dot_general(
            x, wt_ref[...].astype(jnp.bfloat16), _DN,
            preferred_element_type=jnp.float32)
        logits_g = jax.lax.dot_general(
            g_ref[...].astype(jnp.bfloat16),
            wgene_ref[...].astype(jnp.bfloat16), _DN,
            preferred_element_type=jnp.float32)
        logits = (logits_h + logits_g / TEMP) / (1.0 + 1.0 / TEMP)
        m = jnp.max(logits, axis=1, keepdims=True)
        ex = jnp.exp(logits - m)
        scores = ex / jnp.sum(ex, axis=1, keepdims=True)
        lane = jax.lax.broadcasted_iota(jnp.int32, (S2, E), 1)
        m1 = jnp.max(scores, axis=1, keepdims=True)
        i1 = jnp.min(jnp.where(scores == m1, lane, E), axis=1, keepdims=True)
        masked = jnp.where(lane == i1, -jnp.inf, scores)
        m2 = jnp.max(masked, axis=1, keepdims=True)
        i2 = jnp.min(jnp.where(masked == m2, lane, E), axis=1, keepdims=True)
        denom = m1 + m2 + 1e-20
        w1 = m1 / denom
        w2 = m2 / denom
        lane16 = jax.lax.broadcasted_iota(jnp.int32, (S2, 16), 1)
        w9_scr[t] = (jnp.where(lane16 == i1, w1, 0.0)
                     + jnp.where(lane16 == i2, w2, 0.0))

        hsh = _dmlp(x, Wsg_ref[...].astype(jnp.bfloat16),
                    Wsu_ref[...].astype(jnp.bfloat16))
        acc_scr[t] = jax.lax.dot_general(
            hsh.astype(jnp.bfloat16), Wsd_ref[...].astype(jnp.bfloat16),
            _DN, preferred_element_type=jnp.float32)

    lane16 = jax.lax.broadcasted_iota(jnp.int32, (S2, 16), 1)
    xb = xb_scr[t]
    w9 = w9_scr[t]
    yes = []
    for j in range(PAIR):
        e = p * PAIR + j
        wcol = jnp.sum(jnp.where(lane16 == e, w9, 0.0),
                       axis=1, keepdims=True)
        h = _dmlp(xb, Wg_ref[j].astype(jnp.bfloat16),
                  Wu_ref[j].astype(jnp.bfloat16)) * wcol
        yes.append(jax.lax.dot_general(
            h.astype(jnp.bfloat16), Wd_ref[j].astype(jnp.bfloat16),
            _DN, preferred_element_type=jnp.float32))
    total = acc_scr[t] + yes[0] + yes[1]

    @pl.when(p == NP - 1)
    def _emit():
        out_ref[...] = total

    @pl.when(p != NP - 1)
    def _keep():
        acc_scr[t] = total


@jax.jit
def kernel(hidden_states, g, weight_token, weight_gene,
           Wg, Wu, Wd, Wsg, Wsu, Wsd):
    y = pl.pallas_call(
        _moe_body,
        grid=(NP, NT),
        in_specs=[
            pl.BlockSpec((S2, H), lambda p, t: (jnp.where(p == 0, t, 1), 0)),
            pl.BlockSpec((1, G), lambda p, t: (0, 0)),
            pl.BlockSpec((E, H), lambda p, t: (0, 0)),
            pl.BlockSpec((E, G), lambda p, t: (0, 0)),
            pl.BlockSpec((I, H), lambda p, t: (0, 0)),
            pl.BlockSpec((I, H), lambda p, t: (0, 0)),
            pl.BlockSpec((H, I), lambda p, t: (0, 0)),
            pl.BlockSpec((PAIR, I, H), lambda p, t: (p, 0, 0)),
            pl.BlockSpec((PAIR, I, H), lambda p, t: (p, 0, 0)),
            pl.BlockSpec((PAIR, H, I), lambda p, t: (p, 0, 0)),
        ],
        out_specs=pl.BlockSpec(
            (S2, H), lambda p, t: (jnp.where(p == NP - 1, t, 0), 0)),
        out_shape=jax.ShapeDtypeStruct((S, H), jnp.float32),
        scratch_shapes=[pltpu.VMEM((NT, S2, 16), jnp.float32),
                        pltpu.VMEM((NT, S2, H), jnp.bfloat16),
                        pltpu.VMEM((NT, S2, H), jnp.float32)],
        compiler_params=pltpu.CompilerParams(
            dimension_semantics=("arbitrary", "arbitrary")),
    )(hidden_states.reshape(S, H), g, weight_token, weight_gene,
      Wsg, Wsu, Wsd, Wg, Wu, Wd)
    return y.reshape(B, S, H)


# speedup vs baseline: 1.0079x; 1.0079x over previous
import jax
import jax.numpy as jnp
from jax.experimental import pallas as pl
from jax.experimental.pallas import tpu as pltpu

B, S, H = 1, 2048, 1024
E, K = 8, 2
I = 256
G = 128
TEMP = 1.0
PAIR = 2
NP = E // PAIR

_DN = (((1,), (1,)), ((), ()))


def _dmlp(x, wg, wu):
    xg = jax.lax.dot_general(x, wg, _DN, preferred_element_type=jnp.float32)
    xu = jax.lax.dot_general(x, wu, _DN, preferred_element_type=jnp.float32)
    return jnp.where(xg >= 0, xg, 0.01 * xg) * xu


def _moe_body(x_ref, g_ref, wt_ref, wgene_ref, Wsg_ref, Wsu_ref, Wsd_ref,
              Wg_ref, Wu_ref, Wd_ref, out_ref, w9_scr, xb_scr):
    p = pl.program_id(0)

    @pl.when(p == 0)
    def _router_and_shared():
        xb_scr[...] = x_ref[...].astype(jnp.bfloat16)
        x = xb_scr[...]
        logits_h = jax.lax.dot_general(
            x, wt_ref[...].astype(jnp.bfloat16), _DN,
            preferred_element_type=jnp.float32)
        logits_g = jax.lax.dot_general(
            g_ref[...].astype(jnp.bfloat16),
            wgene_ref[...].astype(jnp.bfloat16), _DN,
            preferred_element_type=jnp.float32)
        logits = (logits_h + logits_g / TEMP) / (1.0 + 1.0 / TEMP)
        m = jnp.max(logits, axis=1, keepdims=True)
        ex = jnp.exp(logits - m)
        scores = ex / jnp.sum(ex, axis=1, keepdims=True)
        lane = jax.lax.broadcasted_iota(jnp.int32, (S, E), 1)
        m1 = jnp.max(scores, axis=1, keepdims=True)
        i1 = jnp.min(jnp.where(scores == m1, lane, E), axis=1, keepdims=True)
        masked = jnp.where(lane == i1, -jnp.inf, scores)
        m2 = jnp.max(masked, axis=1, keepdims=True)
        i2 = jnp.min(jnp.where(masked == m2, lane, E), axis=1, keepdims=True)
        denom = m1 + m2 + 1e-20
        w1 = m1 / denom
        w2 = m2 / denom
        lane16 = jax.lax.broadcasted_iota(jnp.int32, (S, 16), 1)
        w9_scr[...] = (jnp.where(lane16 == i1, w1, 0.0)
                       + jnp.where(lane16 == i2, w2, 0.0))

        hsh = _dmlp(x, Wsg_ref[...].astype(jnp.bfloat16),
                    Wsu_ref[...].astype(jnp.bfloat16))
        out_ref[...] = jax.lax.dot_general(
            hsh.astype(jnp.bfloat16), Wsd_ref[...].astype(jnp.bfloat16),
            _DN, preferred_element_type=jnp.float32)

    lane16 = jax.lax.broadcasted_iota(jnp.int32, (S, 16), 1)
    xb = xb_scr[...]
    w9 = w9_scr[...]
    yes = []
    for j in range(PAIR):
        e = p * PAIR + j
        wcol = jnp.sum(jnp.where(lane16 == e, w9, 0.0),
                       axis=1, keepdims=True)
        h = _dmlp(xb, Wg_ref[j].astype(jnp.bfloat16),
                  Wu_ref[j].astype(jnp.bfloat16)) * wcol
        yes.append(jax.lax.dot_general(
            h.astype(jnp.bfloat16), Wd_ref[j].astype(jnp.bfloat16),
            _DN, preferred_element_type=jnp.float32))
    out_ref[...] += yes[0] + yes[1]


@jax.jit
def kernel(hidden_states, g, weight_token, weight_gene,
           Wg, Wu, Wd, Wsg, Wsu, Wsd):
    y = pl.pallas_call(
        _moe_body,
        grid=(NP,),
        in_specs=[
            pl.BlockSpec((S, H), lambda p: (0, 0)),
            pl.BlockSpec((1, G), lambda p: (0, 0)),
            pl.BlockSpec((E, H), lambda p: (0, 0)),
            pl.BlockSpec((E, G), lambda p: (0, 0)),
            pl.BlockSpec((I, H), lambda p: (0, 0)),
            pl.BlockSpec((I, H), lambda p: (0, 0)),
            pl.BlockSpec((H, I), lambda p: (0, 0)),
            pl.BlockSpec((PAIR, I, H), lambda p: (p, 0, 0)),
            pl.BlockSpec((PAIR, I, H), lambda p: (p, 0, 0)),
            pl.BlockSpec((PAIR, H, I), lambda p: (p, 0, 0)),
        ],
        out_specs=pl.BlockSpec((S, H), lambda p: (0, 0)),
        out_shape=jax.ShapeDtypeStruct((S, H), jnp.float32),
        scratch_shapes=[pltpu.VMEM((S, 16), jnp.float32),
                        pltpu.VMEM((S, H), jnp.bfloat16)],
        compiler_params=pltpu.CompilerParams(
            dimension_semantics=("arbitrary",)),
    )(hidden_states.reshape(S, H), g, weight_token, weight_gene,
      Wsg, Wsu, Wsd, Wg, Wu, Wd)
    return y.reshape(B, S, H)
